# baseline (speedup 1.0000x reference)
import functools

import jax
import jax.numpy as jnp
from jax import lax
from jax.experimental import pallas as pl
from jax.experimental.pallas import tpu as pltpu
from jax.experimental.pallas import tpu_sc as plsc

_SHAPE = (4096, 4096)
_P = 12
_XD = 128 - 4 * _P
_NW = 32
_NSPLIT = 2
_PH = _P // _NSPLIT
_TPW = _PH * 32 // _NW


def _slab_transpose(in_ref, out_ref):
    out_ref[...] = in_ref[...].T


def _tc_direct(data):
    return pl.pallas_call(
        _slab_transpose,
        grid=(_XD,),
        in_specs=[pl.BlockSpec((4096, 32), lambda x: (x, 0))],
        out_specs=pl.BlockSpec((32, 4096), lambda x: (x, 0)),
        out_shape=jax.ShapeDtypeStruct(_SHAPE, jnp.float32),
    )(data)


def _slab_transpose_fill(in_ref, part_ref, out_ref):
    del part_ref
    out_ref[...] = in_ref[...].T


def _tc_direct_fill(data, part):
    return pl.pallas_call(
        _slab_transpose_fill,
        grid=(_XD,),
        in_specs=[
            pl.BlockSpec((4096, 32), lambda x: (x, 0)),
            pl.BlockSpec(memory_space=pl.ANY),
        ],
        out_specs=pl.BlockSpec((32, 4096), lambda x: (x, 0)),
        out_shape=jax.ShapeDtypeStruct(_SHAPE, jnp.float32),
        input_output_aliases={1: 0},
    )(data, part)


def _sc_repack(data, stage):
    mesh = plsc.VectorSubcoreMesh(core_axis_name="c", subcore_axis_name="s")
    x0_base = (128 - 4 * _P) // 4 + stage * _PH

    @functools.partial(
        pl.kernel,
        mesh=mesh,
        out_type=jax.ShapeDtypeStruct((_PH * 32, 128, 128), jnp.float32),
        scratch_types=[
            pltpu.VMEM((128, 32), jnp.float32),
            pltpu.VMEM((128, 32), jnp.float32),
            pltpu.VMEM((128, 32), jnp.float32),
            pltpu.VMEM((128, 32), jnp.float32),
            pltpu.VMEM((128, 128), jnp.float32),
        ],
    )
    def sc_kernel(data_hbm, s3_hbm, i0, i1, i2, i3, outbuf):
        wid = lax.axis_index("s") * 2 + lax.axis_index("c")
        inbufs = (i0, i1, i2, i3)
        for t in range(_TPW):
            flat = wid * _TPW + t
            x0 = x0_base + flat // 32
            y0 = flat % 32
            for xi in range(4):
                x = x0 * 4 + xi
                pltpu.sync_copy(
                    data_hbm.at[pl.ds(x * 4096 + y0 * 128, 128)], inbufs[xi])

            def repack(r, carry):
                for xi in range(4):
                    outbuf[r, pl.ds(xi * 32, 16)] = inbufs[xi][r, pl.ds(0, 16)]
                    outbuf[r, pl.ds(xi * 32 + 16, 16)] = (
                        inbufs[xi][r, pl.ds(16, 16)])
                return carry

            lax.fori_loop(0, 128, repack, 0)
            pltpu.sync_copy(outbuf, s3_hbm.at[flat])

    return sc_kernel(data)


def _tile_merge(s3_ref, part_ref, out_ref):
    del part_ref
    out_ref[...] = s3_ref[...].transpose(2, 0, 1).reshape(128, 4096)


def _tile_merge_first(s3_ref, out_ref):
    out_ref[...] = s3_ref[...].transpose(2, 0, 1).reshape(128, 4096)


def _tc_merge(s3, part, stage):
    base = (128 - 4 * _P) // 4 + stage * _PH
    if part is None:
        return pl.pallas_call(
            _tile_merge_first,
            grid=(_PH,),
            in_specs=[pl.BlockSpec((32, 128, 128), lambda j: (j, 0, 0))],
            out_specs=pl.BlockSpec((128, 4096), lambda j: (base + j, 0)),
            out_shape=jax.ShapeDtypeStruct(_SHAPE, jnp.float32),
        )(s3)
    return pl.pallas_call(
        _tile_merge,
        grid=(_PH,),
        in_specs=[
            pl.BlockSpec((32, 128, 128), lambda j: (j, 0, 0)),
            pl.BlockSpec(memory_space=pl.ANY),
        ],
        out_specs=pl.BlockSpec((128, 4096), lambda j: (base + j, 0)),
        out_shape=jax.ShapeDtypeStruct(_SHAPE, jnp.float32),
        input_output_aliases={1: 0},
    )(s3, part)


def kernel(block_mask, data):
    del block_mask
    stages = [_sc_repack(data, h) for h in range(_NSPLIT)]
    out = None
    for h in range(_NSPLIT):
        out = _tc_merge(stages[h], out, h)
    return _tc_direct_fill(data, out)


# speedup vs baseline: 1.2219x; 1.2219x over previous
import functools

import jax
import jax.numpy as jnp
from jax import lax
from jax.experimental import pallas as pl
from jax.experimental.pallas import tpu as pltpu
from jax.experimental.pallas import tpu_sc as plsc

_SHAPE = (4096, 4096)
_P = 12
_XD = 128 - 4 * _P
_NW = 32
_NSPLIT = 1
_PH = _P // _NSPLIT
_TPW = _PH * 32 // _NW


def _slab_transpose(in_ref, out_ref):
    out_ref[...] = in_ref[...].T


def _tc_direct(data):
    return pl.pallas_call(
        _slab_transpose,
        grid=(_XD,),
        in_specs=[pl.BlockSpec((4096, 32), lambda x: (x, 0))],
        out_specs=pl.BlockSpec((32, 4096), lambda x: (x, 0)),
        out_shape=jax.ShapeDtypeStruct(_SHAPE, jnp.float32),
    )(data)


def _slab_transpose_fill(in_ref, part_ref, out_ref):
    del part_ref
    out_ref[...] = in_ref[...].T


def _tc_direct_fill(data, part):
    return pl.pallas_call(
        _slab_transpose_fill,
        grid=(_XD,),
        in_specs=[
            pl.BlockSpec((4096, 32), lambda x: (x, 0)),
            pl.BlockSpec(memory_space=pl.ANY),
        ],
        out_specs=pl.BlockSpec((32, 4096), lambda x: (x, 0)),
        out_shape=jax.ShapeDtypeStruct(_SHAPE, jnp.float32),
        input_output_aliases={1: 0},
    )(data, part)


def _sc_repack(data, stage):
    mesh = plsc.VectorSubcoreMesh(core_axis_name="c", subcore_axis_name="s")
    x0_base = (128 - 4 * _P) // 4 + stage * _PH

    @functools.partial(
        pl.kernel,
        mesh=mesh,
        out_type=jax.ShapeDtypeStruct((_PH * 32, 128, 128), jnp.float32),
        scratch_types=[
            pltpu.VMEM((128, 32), jnp.float32),
            pltpu.VMEM((128, 32), jnp.float32),
            pltpu.VMEM((128, 32), jnp.float32),
            pltpu.VMEM((128, 32), jnp.float32),
            pltpu.VMEM((128, 128), jnp.float32),
        ],
    )
    def sc_kernel(data_hbm, s3_hbm, i0, i1, i2, i3, outbuf):
        wid = lax.axis_index("s") * 2 + lax.axis_index("c")
        inbufs = (i0, i1, i2, i3)
        for t in range(_TPW):
            flat = wid * _TPW + t
            x0 = x0_base + flat // 32
            y0 = flat % 32
            for xi in range(4):
                x = x0 * 4 + xi
                pltpu.sync_copy(
                    data_hbm.at[pl.ds(x * 4096 + y0 * 128, 128)], inbufs[xi])

            def repack(r, carry):
                for xi in range(4):
                    outbuf[r, pl.ds(xi * 32, 16)] = inbufs[xi][r, pl.ds(0, 16)]
                    outbuf[r, pl.ds(xi * 32 + 16, 16)] = (
                        inbufs[xi][r, pl.ds(16, 16)])
                return carry

            lax.fori_loop(0, 128, repack, 0)
            pltpu.sync_copy(outbuf, s3_hbm.at[flat])

    return sc_kernel(data)


def _tile_merge(s3_ref, part_ref, out_ref):
    del part_ref
    out_ref[...] = s3_ref[...].transpose(2, 0, 1).reshape(128, 4096)


def _tile_merge_first(s3_ref, out_ref):
    out_ref[...] = s3_ref[...].transpose(2, 0, 1).reshape(128, 4096)


def _tc_merge(s3, part, stage):
    base = (128 - 4 * _P) // 4 + stage * _PH
    if part is None:
        return pl.pallas_call(
            _tile_merge_first,
            grid=(_PH,),
            in_specs=[pl.BlockSpec((32, 128, 128), lambda j: (j, 0, 0))],
            out_specs=pl.BlockSpec((128, 4096), lambda j: (base + j, 0)),
            out_shape=jax.ShapeDtypeStruct(_SHAPE, jnp.float32),
        )(s3)
    return pl.pallas_call(
        _tile_merge,
        grid=(_PH,),
        in_specs=[
            pl.BlockSpec((32, 128, 128), lambda j: (j, 0, 0)),
            pl.BlockSpec(memory_space=pl.ANY),
        ],
        out_specs=pl.BlockSpec((128, 4096), lambda j: (base + j, 0)),
        out_shape=jax.ShapeDtypeStruct(_SHAPE, jnp.float32),
        input_output_aliases={1: 0},
    )(s3, part)


def kernel(block_mask, data):
    del block_mask
    part = _tc_direct(data)
    s3 = _sc_repack(data, 0)
    return _tc_merge(s3, part, 0)
